# Initial kernel scaffold; baseline (speedup 1.0000x reference)
import jax
import jax.numpy as jnp
from jax.experimental import pallas as pl


def kernel(entity_emb, relation_emb, type_emb, a1_0, b1_0, a2_0, b2_0, a1_1, b1_1, a2_1, b2_1, W, a1_o, b1_o, a2_o, b2_o, W_entities, W_types, edge_list, edge_type, batch_inputs):
    raise NotImplementedError("write your pallas kernel here")


# speedup vs baseline: 1.5860x; 1.5860x over previous
import functools

import jax
import jax.numpy as jnp
from jax import lax
from jax.experimental import pallas as pl
from jax.experimental.pallas import tpu as pltpu
from jax.experimental.pallas import tpu_sc as plsc

NUM_ENT = 10000
NUM_TYPE = 1000
NUM_REL = 500
D_IN = 128
E = 160000
ALPHA = 0.2

NS_PAD = 10240
ND_PAD = 1024
NR_PAD = 512
E_PAD = 163840
ACC_W = 80
CHUNK = 32
EDGES_PER_TILE = E_PAD // 32
CHUNKS_PER_TILE = EDGES_PER_TILE // CHUNK

_f32 = jnp.float32
_i32 = jnp.int32

_GDN = lax.GatherDimensionNumbers(
    offset_dims=(), collapsed_slice_dims=(0,), start_index_map=(0,))


def _sc_mesh():
    return plsc.VectorSubcoreMesh(core_axis_name="c", subcore_axis_name="s")


def _prep_nodes(x_pad, wcat, npad, bm):
    def body(x_ref, w_ref, xn_ref, s0_ref, s1_ref, s2_ref, s3_ref):
        x = x_ref[...]
        nrm = jnp.sqrt(jnp.sum(x * x, axis=1, keepdims=True))
        xn = x / jnp.maximum(nrm, 1e-12)
        y = jnp.dot(xn, w_ref[...], preferred_element_type=_f32)
        xn_ref[...] = xn
        s0_ref[...] = y[:, 0:160]
        s1_ref[...] = y[:, 160:320]
        s2_ref[...] = y[:, 320:480]
        s3_ref[...] = y[:, 480:640]

    tab = lambda: pl.BlockSpec((bm, 160), lambda i: (i, 0))
    return pl.pallas_call(
        body,
        grid=(npad // bm,),
        in_specs=[pl.BlockSpec((bm, 128), lambda i: (i, 0)),
                  pl.BlockSpec((128, 640), lambda i: (0, 0))],
        out_specs=[pl.BlockSpec((bm, 128), lambda i: (i, 0)),
                   tab(), tab(), tab(), tab()],
        out_shape=[jax.ShapeDtypeStruct((npad, 128), _f32)]
        + [jax.ShapeDtypeStruct((npad, 160), _f32)] * 4,
    )(x_pad, wcat)


def _prep_rel(rel_pad, w1, wW, w2):
    def body(r_ref, w1_ref, wW_ref, w2_ref, orel_ref,
             r0_ref, r1_ref, r2_ref, r3_ref,
             q0_ref, q1_ref, q2_ref, q3_ref):
        r = r_ref[...]
        y1 = jnp.dot(r, w1_ref[...], preferred_element_type=_f32)
        orel = jnp.dot(r, wW_ref[...], preferred_element_type=_f32)
        y2 = jnp.dot(orel, w2_ref[...], preferred_element_type=_f32)
        orel_ref[...] = orel
        r0_ref[...] = y1[:, 0:160]
        r1_ref[...] = y1[:, 160:320]
        r2_ref[...] = y1[:, 320:480]
        r3_ref[...] = y1[:, 480:640]
        q0_ref[...] = y2[:, 0:160]
        q1_ref[...] = y2[:, 160:320]
        q2_ref[...] = y2[:, 320:480]
        q3_ref[...] = y2[:, 480:640]

    n = NR_PAD
    return pl.pallas_call(
        body,
        out_shape=[jax.ShapeDtypeStruct((n, 256), _f32)]
        + [jax.ShapeDtypeStruct((n, 160), _f32)] * 8,
    )(rel_pad, w1, wW, w2)


def _elu(x):
    return jnp.where(x > 0, x, jnp.exp(x) - 1.0)


def _sum_acc(a):
    num = a[0, :, :64] + a[1, :, :64]
    den = a[0, :, 64:65] + a[1, :, 64:65]
    return num, den


def _mid(e0lo, e0hi, e1lo, e1hi, wq, npad, bm):
    def body(a0l_ref, a0h_ref, a1l_ref, a1h_ref, w_ref,
             q0_ref, q1_ref, q2_ref, q3_ref):
        n0l, d0 = _sum_acc(a0l_ref[...])
        n0h, _ = _sum_acc(a0h_ref[...])
        n1l, d1 = _sum_acc(a1l_ref[...])
        n1h, _ = _sum_acc(a1h_ref[...])
        d0 = jnp.where(d0 == 0.0, 1e-12, d0)
        d1 = jnp.where(d1 == 0.0, 1e-12, d1)
        h = jnp.concatenate([_elu(jnp.concatenate([n0l, n0h], axis=1) / d0),
                             _elu(jnp.concatenate([n1l, n1h], axis=1) / d1)],
                            axis=1)
        y = jnp.dot(h, w_ref[...], preferred_element_type=_f32)
        q0_ref[...] = y[:, 0:160]
        q1_ref[...] = y[:, 160:320]
        q2_ref[...] = y[:, 320:480]
        q3_ref[...] = y[:, 480:640]

    acc = lambda: pl.BlockSpec((2, bm, ACC_W), lambda i: (0, i, 0))
    tab = lambda: pl.BlockSpec((bm, 160), lambda i: (i, 0))
    return pl.pallas_call(
        body,
        grid=(npad // bm,),
        in_specs=[acc(), acc(), acc(), acc(),
                  pl.BlockSpec((256, 640), lambda i: (0, 0))],
        out_specs=[tab(), tab(), tab(), tab()],
        out_shape=[jax.ShapeDtypeStruct((npad, 160), _f32)] * 4,
    )(e0lo, e0hi, e1lo, e1hi, wq)


def _final(xn, a0, a1, a2, a3, mask, wout, npad, bm):
    def body(x_ref, a0_ref, a1_ref, a2_ref, a3_ref, m_ref, w_ref, o_ref):
        n0, d = _sum_acc(a0_ref[...])
        n1, _ = _sum_acc(a1_ref[...])
        n2, _ = _sum_acc(a2_ref[...])
        n3, _ = _sum_acc(a3_ref[...])
        d = jnp.where(d == 0.0, 1e-12, d)
        g = _elu(jnp.concatenate([n0, n1, n2, n3], axis=1) / d)
        v = jnp.dot(x_ref[...], w_ref[...], preferred_element_type=_f32)
        v = v + m_ref[...][:, 0:1] * g
        nrm = jnp.sqrt(jnp.sum(v * v, axis=1, keepdims=True))
        o_ref[...] = v / jnp.maximum(nrm, 1e-12)

    acc = lambda: pl.BlockSpec((2, bm, ACC_W), lambda i: (0, i, 0))
    return pl.pallas_call(
        body,
        grid=(npad // bm,),
        in_specs=[pl.BlockSpec((bm, 128), lambda i: (i, 0)),
                  acc(), acc(), acc(), acc(),
                  pl.BlockSpec((bm, 16), lambda i: (i, 0)),
                  pl.BlockSpec((128, 256), lambda i: (0, 0))],
        out_specs=pl.BlockSpec((bm, 256), lambda i: (i, 0)),
        out_shape=jax.ShapeDtypeStruct((npad, 256), _f32),
    )(xn, a0, a1, a2, a3, mask, wout)


def _edge_pass(S, Dd, Rr, idx_packed):
    ent_rpt = NS_PAD // 16
    typ_rpt = ND_PAD // 16

    @functools.partial(
        pl.kernel,
        out_type=[jax.ShapeDtypeStruct((2 * NS_PAD, ACC_W), _f32),
                  jax.ShapeDtypeStruct((2 * ND_PAD, ACC_W), _f32)],
        mesh=_sc_mesh(),
        compiler_params=pltpu.CompilerParams(use_tc_tiling_on_sc=False),
        scratch_types=[
            pltpu.VMEM((3, CHUNK), _i32),
            pltpu.VMEM((3, CHUNK), _i32),
            pltpu.VMEM((3, CHUNK), _i32),
            pltpu.VMEM((3, CHUNK), _i32),
            pltpu.VMEM((CHUNK, 160), _f32),
            pltpu.VMEM((CHUNK, 160), _f32),
            pltpu.VMEM((CHUNK, 160), _f32),
            pltpu.VMEM((CHUNK, 160), _f32),
            pltpu.VMEM((CHUNK, 160), _f32),
            pltpu.VMEM((CHUNK, 160), _f32),
            pltpu.VMEM((CHUNK, ACC_W), _f32),
            pltpu.VMEM((CHUNK, ACC_W), _f32),
            pltpu.VMEM_SHARED((NS_PAD, ACC_W), _f32),
            pltpu.VMEM_SHARED((ND_PAD, ACC_W), _f32),
            pltpu.SemaphoreType.DMA,
            pltpu.SemaphoreType.DMA,
            pltpu.SemaphoreType.DMA,
        ],
    )
    def k(S_h, D_h, R_h, idx_h, ent_out, typ_out,
          idx_0, idx_1, idx_2, idx_3,
          rs_a, rd_a, rr_a, rs_b, rd_b, rr_b,
          oe, ot, ent_acc, typ_acc, sem_a, sem_b, sem_sc):
        c = lax.axis_index("c")
        s = lax.axis_index("s")
        wid = c * 16 + s
        zf16 = jnp.zeros((16,), _f32)

        @pl.loop(0, CHUNK)
        def _zero_oe(i):
            for kk in range(ACC_W // 16):
                oe[i, pl.ds(kk * 16, 16)] = zf16

        for t in range(ent_rpt // CHUNK):
            pltpu.sync_copy(oe, ent_acc.at[pl.ds(s * ent_rpt + t * CHUNK,
                                                 CHUNK)])
        for t in range(typ_rpt // CHUNK):
            pltpu.sync_copy(oe, typ_acc.at[pl.ds(s * typ_rpt + t * CHUNK,
                                                 CHUNK)])
        plsc.subcore_barrier()

        def issue(j, idxb, rs, rd, rr, sem):
            cid = wid * CHUNKS_PER_TILE + j
            pltpu.sync_copy(idx_h.at[cid], idxb)
            pltpu.async_copy(S_h.at[idxb.at[0]], rs, sem)
            pltpu.async_copy(D_h.at[idxb.at[1]], rd, sem)
            pltpu.async_copy(R_h.at[idxb.at[2]], rr, sem)

        def drain(rs, rd, rr, sem):
            pltpu.make_async_copy(S_h.at[pl.ds(0, CHUNK)], rs, sem).wait()
            pltpu.make_async_copy(D_h.at[pl.ds(0, CHUNK)], rd, sem).wait()
            pltpu.make_async_copy(R_h.at[pl.ds(0, CHUNK)], rr, sem).wait()

        def compute(idxb, rs, rd, rr):
            @pl.loop(0, CHUNK)
            def _edge(i):
                z1 = (rs[i, pl.ds(128, 16)] + rd[i, pl.ds(128, 16)]
                      + rr[i, pl.ds(128, 16)])
                z2 = (rs[i, pl.ds(144, 16)] + rd[i, pl.ds(144, 16)]
                      + rr[i, pl.ds(144, 16)])
                w1 = jnp.exp(-jnp.where(z1 >= 0, z1, ALPHA * z1))
                w2 = jnp.exp(-jnp.where(z2 >= 0, z2, ALPHA * z2))
                for kk in range(4):
                    m = (rs[i, pl.ds(kk * 16, 16)]
                         + rd[i, pl.ds(kk * 16, 16)]
                         + rr[i, pl.ds(kk * 16, 16)])
                    oe[i, pl.ds(kk * 16, 16)] = m * w1
                for kk in range(4):
                    m = (rs[i, pl.ds(64 + kk * 16, 16)]
                         + rd[i, pl.ds(64 + kk * 16, 16)]
                         + rr[i, pl.ds(64 + kk * 16, 16)])
                    ot[i, pl.ds(kk * 16, 16)] = m * w2
                oe[i, pl.ds(64, 16)] = w1
                ot[i, pl.ds(64, 16)] = w2

            pltpu.async_copy(oe, ent_acc.at[idxb.at[0]], sem_sc, add=True)
            pltpu.async_copy(ot, typ_acc.at[idxb.at[1]], sem_sc, add=True)

        idxs = (idx_0, idx_1, idx_2, idx_3)
        rows = ((rs_a, rd_a, rr_a, sem_a), (rs_b, rd_b, rr_b, sem_b))

        def drain_sc():
            pltpu.make_async_copy(oe, ent_acc.at[idx_0.at[0]], sem_sc).wait()
            pltpu.make_async_copy(ot, typ_acc.at[idx_0.at[1]], sem_sc).wait()

        issue(0, idx_0, rs_a, rd_a, rr_a, sem_a)

        @pl.loop(0, CHUNKS_PER_TILE // 4)
        def _outer(h):
            j = h * 4
            for p in range(4):
                jp = j + p
                rs, rd, rr, sem = rows[p % 2]
                rs2, rd2, rr2, sem2 = rows[(p + 1) % 2]
                issue(jp + 1, idxs[(p + 1) % 4], rs2, rd2, rr2, sem2)
                drain(rs, rd, rr, sem)

                @pl.when(jp > 0)
                def _():
                    drain_sc()

                compute(idxs[p], rs, rd, rr)

        drain_sc()
        drain(rs_a, rd_a, rr_a, sem_a)
        plsc.subcore_barrier()

        rb = s * ent_rpt
        pltpu.sync_copy(ent_acc.at[pl.ds(rb, ent_rpt)],
                        ent_out.at[pl.ds(c * NS_PAD + rb, ent_rpt)])
        rb = s * typ_rpt
        pltpu.sync_copy(typ_acc.at[pl.ds(rb, typ_rpt)],
                        typ_out.at[pl.ds(c * ND_PAD + rb, typ_rpt)])

    ent, typ = k(S, Dd, Rr, idx_packed)
    return ent.reshape(2, NS_PAD, ACC_W), typ.reshape(2, ND_PAD, ACC_W)


def _masks(b_ent, b_typ):
    @functools.partial(
        pl.kernel,
        out_type=[jax.ShapeDtypeStruct((NS_PAD, 16), _f32),
                  jax.ShapeDtypeStruct((ND_PAD, 16), _f32)],
        mesh=_sc_mesh(),
        compiler_params=pltpu.CompilerParams(use_tc_tiling_on_sc=False),
        scratch_types=[
            pltpu.VMEM((256,), _i32),
            pltpu.VMEM((256, 16), _f32),
            pltpu.VMEM_SHARED((NS_PAD, 16), _f32),
            pltpu.VMEM_SHARED((ND_PAD, 16), _f32),
        ],
    )
    def k(be_h, bt_h, me_h, mt_h, idxb, ones_b, msh_e, msh_t):
        c = lax.axis_index("c")
        s = lax.axis_index("s")
        zf16 = jnp.zeros((16,), _f32)
        of16 = jnp.ones((16,), _f32)

        @pl.loop(0, 64)
        def _z(i):
            ones_b[i, pl.ds(0, 16)] = zf16

        erpt = NS_PAD // 16
        trpt = ND_PAD // 16
        for t in range(erpt // 64):
            pltpu.sync_copy(ones_b.at[pl.ds(0, 64)],
                            msh_e.at[pl.ds(s * erpt + t * 64, 64)])
        pltpu.sync_copy(ones_b.at[pl.ds(0, trpt)],
                        msh_t.at[pl.ds(s * trpt, trpt)])

        @pl.loop(0, 256)
        def _o(i):
            ones_b[i, pl.ds(0, 16)] = of16

        plsc.subcore_barrier()
        pltpu.sync_copy(be_h.at[pl.ds(s * 256, 256)], idxb)
        pltpu.sync_copy(ones_b, msh_e.at[idxb])
        pltpu.sync_copy(bt_h.at[pl.ds(s * 256, 256)], idxb)
        pltpu.sync_copy(ones_b, msh_t.at[idxb])
        plsc.subcore_barrier()

        @pl.when(c == 0)
        def _():
            pltpu.sync_copy(msh_e.at[pl.ds(s * erpt, erpt)],
                            me_h.at[pl.ds(s * erpt, erpt)])
            pltpu.sync_copy(msh_t.at[pl.ds(s * trpt, trpt)],
                            mt_h.at[pl.ds(s * trpt, trpt)])

    return k(b_ent, b_typ)


def kernel(entity_emb, relation_emb, type_emb, a1_0, b1_0, a2_0, b2_0,
           a1_1, b1_1, a2_1, b2_1, W, a1_o, b1_o, a2_o, b2_o,
           W_entities, W_types, edge_list, edge_type, batch_inputs):
    D = D_IN

    ent_pad = jnp.pad(entity_emb, ((0, NS_PAD - NUM_ENT), (0, 0)))
    typ_pad = jnp.pad(type_emb, ((0, ND_PAD - NUM_TYPE), (0, 0)))
    rel_pad = jnp.pad(relation_emb, ((0, NR_PAD - NUM_REL), (0, 0)))
    npad_e = E_PAD + CHUNK - E
    src_p = jnp.concatenate([edge_list[0], jnp.full((npad_e,), NUM_ENT, _i32)])
    dst_p = jnp.concatenate([edge_list[1], jnp.full((npad_e,), NUM_TYPE, _i32)])
    et_p = jnp.concatenate([edge_type, jnp.full((npad_e,), NUM_REL, _i32)])
    idx_packed = jnp.stack([src_p.reshape(-1, CHUNK),
                            dst_p.reshape(-1, CHUNK),
                            et_p.reshape(-1, CHUNK)], axis=1)
    b_ent = jnp.asarray(batch_inputs[:, 0], _i32)
    b_typ = jnp.asarray(batch_inputs[:, 2], _i32)

    def wcat_for(seg):
        lo, hi = seg
        A10 = a1_0[:, lo:hi].T
        A20 = a2_0[:, lo:hi].T
        A11 = a1_1[:, lo:hi].T
        A21 = a2_1[:, lo:hi].T
        rep = lambda v: jnp.tile(v[:, None], (1, 16))
        t0a, t0b = rep(A10 @ b1_0[0]), rep(A20 @ b2_0[0])
        t1a, t1b = rep(A11 @ b1_1[0]), rep(A21 @ b2_1[0])
        return jnp.concatenate([
            A10[:, :64], A20[:, :64], t0a, t0b,
            A10[:, 64:], A20[:, 64:], t0a, t0b,
            A11[:, :64], A21[:, :64], t1a, t1b,
            A11[:, 64:], A21[:, 64:], t1a, t1b,
        ], axis=1)

    wcat_ent = wcat_for((0, D))
    wcat_typ = wcat_for((D, 2 * D))
    wcat_rel = wcat_for((2 * D, 3 * D))

    def wq_for(A1, A2):
        W1 = A1.T
        W2 = A2.T
        rep = lambda v: jnp.tile(v[:, None], (1, 16))
        ta, tb = rep(W1 @ b1_o[0]), rep(W2 @ b2_o[0])
        cols = []
        for q in range(4):
            cols += [W1[:, q * 64:(q + 1) * 64], W2[:, q * 64:(q + 1) * 64],
                     ta, tb]
        return jnp.concatenate(cols, axis=1)

    wq_ent = wq_for(a1_o[:, :256], a2_o[:, :256])
    wq_typ = wq_for(a1_o[:, 256:512], a2_o[:, 256:512])
    wq_rel = wq_for(a1_o[:, 512:768], a2_o[:, 512:768])

    x1n, S00, S01, S10, S11 = _prep_nodes(ent_pad, wcat_ent, NS_PAD, 256)
    x2n, D00, D01, D10, D11 = _prep_nodes(typ_pad, wcat_typ, ND_PAD, 256)
    (out_rel, R00, R01, R10, R11,
     Q0, Q1, Q2, Q3) = _prep_rel(rel_pad, wcat_rel, W, wq_rel)
    mask_e, mask_t = _masks(b_ent, b_typ)

    E00, T00 = _edge_pass(S00, D00, R00, idx_packed)
    E01, T01 = _edge_pass(S01, D01, R01, idx_packed)
    E10, T10 = _edge_pass(S10, D10, R10, idx_packed)
    E11, T11 = _edge_pass(S11, D11, R11, idx_packed)

    SQ0, SQ1, SQ2, SQ3 = _mid(E00, E01, E10, E11, wq_ent, NS_PAD, 256)
    DQ0, DQ1, DQ2, DQ3 = _mid(T00, T01, T10, T11, wq_typ, ND_PAD, 256)

    GE, GT = [], []
    for SQ, DQ, RQ in ((SQ0, DQ0, Q0), (SQ1, DQ1, Q1),
                       (SQ2, DQ2, Q2), (SQ3, DQ3, Q3)):
        ge, gt = _edge_pass(SQ, DQ, RQ, idx_packed)
        GE.append(ge)
        GT.append(gt)

    out_e = _final(x1n, GE[0], GE[1], GE[2], GE[3], mask_e, W_entities,
                   NS_PAD, 256)
    out_t = _final(x2n, GT[0], GT[1], GT[2], GT[3], mask_t, W_types,
                   ND_PAD, 256)

    return (out_e[:NUM_ENT], out_t[:NUM_TYPE], out_rel[:NUM_REL])
